# baseline (speedup 1.0000x reference)
import functools

import jax
import jax.numpy as jnp
from jax import lax
from jax.experimental import pallas as pl
from jax.experimental.pallas import tpu as pltpu
from jax.experimental.pallas import tpu_sc as plsc

N_FACTORS = 32
BATCH = 16384
NC = 2
NS = 16
L = 16
NW = NC * NS
BPW = BATCH // NW
CH = 128
NCH = BPW // CH


def _body(users_hbm, items_hbm, ue_hbm, ie_hbm, ub_hbm, ib_hbm, out_hbm,
          uidx_v, iidx_v, ue_v, ie_v, ub_v, ib_v, out_v, sem):
    wid = lax.axis_index("s") * NC + lax.axis_index("c")
    base = wid * BPW

    for i in range(NCH):
        pltpu.sync_copy(users_hbm.at[pl.ds(base + i * CH, CH)], uidx_v.at[i])
        pltpu.sync_copy(items_hbm.at[pl.ds(base + i * CH, CH)], iidx_v.at[i])

    copies = []
    for i in range(NCH):
        sl = pl.ds(i * CH, CH)
        copies.append(pltpu.async_copy(ue_hbm.at[uidx_v.at[i]], ue_v.at[sl], sem))
        copies.append(pltpu.async_copy(ie_hbm.at[iidx_v.at[i]], ie_v.at[sl], sem))
        copies.append(pltpu.async_copy(ub_hbm.at[uidx_v.at[i]], ub_v.at[sl], sem))
        copies.append(pltpu.async_copy(ib_hbm.at[iidx_v.at[i]], ib_v.at[sl], sem))
    for c in copies:
        c.wait()

    lane = lax.iota(jnp.int32, L)

    def group(g, carry):
        rows = g * L + lane
        accs = [None] * 4
        for j in range(N_FACTORS):
            col = jnp.full((L,), j, jnp.int32)
            a = plsc.load_gather(ue_v, [rows, col])
            b = plsc.load_gather(ie_v, [rows, col])
            p = a * b
            k = j % 4
            accs[k] = p if accs[k] is None else accs[k] + p
        bias = ub_v[pl.ds(g * L, L)] + ib_v[pl.ds(g * L, L)]
        out_v[pl.ds(g * L, L)] = (accs[0] + accs[1]) + (accs[2] + accs[3]) + bias
        return carry

    lax.fori_loop(0, BPW // L, group, 0, unroll=2)

    pltpu.sync_copy(out_v, out_hbm.at[pl.ds(base, BPW)])


@jax.jit
def kernel(users, items, user_embeddings, item_embeddings, user_biases, item_biases):
    mesh = plsc.VectorSubcoreMesh(
        core_axis_name="c", subcore_axis_name="s", num_cores=NC, num_subcores=NS
    )
    run = functools.partial(
        pl.kernel,
        out_type=jax.ShapeDtypeStruct((BATCH,), jnp.float32),
        mesh=mesh,
        compiler_params=pltpu.CompilerParams(
            needs_layout_passes=False, use_tc_tiling_on_sc=False
        ),
        scratch_types=[
            pltpu.VMEM((NCH, CH), jnp.int32),
            pltpu.VMEM((NCH, CH), jnp.int32),
            pltpu.VMEM((BPW, N_FACTORS), jnp.float32),
            pltpu.VMEM((BPW, N_FACTORS), jnp.float32),
            pltpu.VMEM((BPW,), jnp.float32),
            pltpu.VMEM((BPW,), jnp.float32),
            pltpu.VMEM((BPW,), jnp.float32),
            pltpu.SemaphoreType.DMA,
        ],
    )(_body)
    return run(users.astype(jnp.int32), items.astype(jnp.int32),
               user_embeddings, item_embeddings,
               user_biases.reshape(-1), item_biases.reshape(-1))
